# baseline (speedup 1.0000x reference)
import numpy as np

import jax
import jax.numpy as jnp
from jax import lax
from jax.experimental import pallas as pl
from jax.experimental.pallas import tpu as pltpu
from jax.experimental.shard_map import shard_map
from jax.sharding import Mesh, PartitionSpec as P

IDX_BLOCK = 8192
UNROLL = 512


def _gather_kernel(idx_ref, table_ref, out_ref):
    def body(c, carry):
        base = c * UNROLL
        rows = [table_ref[idx_ref[0, 0, base + u]] for u in range(UNROLL)]
        for u in range(UNROLL):
            out_ref[base + u] = rows[u]
        return carry

    lax.fori_loop(0, IDX_BLOCK // UNROLL, body, 0)


def _gather(table: jax.Array, flat_idx: jax.Array) -> jax.Array:
    num_rows, dim = table.shape
    n = flat_idx.shape[0]
    n_pad = ((n + IDX_BLOCK - 1) // IDX_BLOCK) * IDX_BLOCK
    num_blocks = n_pad // IDX_BLOCK
    idx3d = jnp.pad(flat_idx, (0, n_pad - n)).reshape(num_blocks, 1, IDX_BLOCK)
    table3 = table.reshape(num_rows, 1, dim)

    itemsize = table.dtype.itemsize
    cost = pl.CostEstimate(
        flops=0,
        transcendentals=0,
        bytes_accessed=num_rows * dim * itemsize + n_pad * dim * itemsize
        + n_pad * 4,
    )

    out = pl.pallas_call(
        _gather_kernel,
        grid=(num_blocks,),
        in_specs=[
            pl.BlockSpec((1, 1, IDX_BLOCK), lambda i: (i, 0, 0),
                         memory_space=pltpu.SMEM),
            pl.BlockSpec((num_rows, 1, dim), lambda i: (0, 0, 0)),
        ],
        out_specs=pl.BlockSpec((IDX_BLOCK, 1, dim), lambda i: (i, 0, 0)),
        out_shape=jax.ShapeDtypeStruct((n_pad, 1, dim), table.dtype),
        compiler_params=pltpu.CompilerParams(
            dimension_semantics=("parallel",),
            vmem_limit_bytes=32 * 1024 * 1024,
        ),
        cost_estimate=cost,
    )(idx3d, table3)

    return out.reshape(n_pad, dim)[:n]


def kernel(table, num_group):
    num_rows, dim = table.shape

    def _local(t, g):
        flat = jnp.clip(g.reshape(-1).astype(jnp.int32), 0, num_rows - 1)
        return _gather(t, flat)

    devs = jax.devices()
    ids = num_group.reshape(-1, 1) if num_group.ndim < 2 else num_group
    if len(devs) >= 2 and ids.shape[0] % 2 == 0:
        mesh = Mesh(np.array(devs[:2]), ("x",))
        gather2 = shard_map(
            _local, mesh=mesh,
            in_specs=(P(), P("x")), out_specs=P("x"), check_rep=False,
        )
        n_chunks = 4 if ids.shape[0] % 8 == 0 else 1
        if n_chunks > 1:
            outs = [gather2(table, c) for c in jnp.split(ids, n_chunks, axis=0)]
            out = jnp.concatenate(outs, axis=0)
        else:
            out = gather2(table, ids)
    else:
        out = _local(table, ids)
    return out.reshape(num_group.shape + (dim,))


# speedup vs baseline: 5.5977x; 5.5977x over previous
import numpy as np

import jax
import jax.numpy as jnp
from jax import lax
from jax.experimental import pallas as pl
from jax.experimental.pallas import tpu as pltpu
from jax.experimental.shard_map import shard_map
from jax.sharding import Mesh, PartitionSpec as P

IDX_BLOCK = 8192
UNROLL = 512


def _gather_kernel(idx_ref, table_ref, out_ref):
    def body(c, carry):
        base = c * UNROLL
        rows = [table_ref[idx_ref[0, 0, base + u]] for u in range(UNROLL)]
        for u in range(UNROLL):
            out_ref[base + u] = rows[u]
        return carry

    lax.fori_loop(0, IDX_BLOCK // UNROLL, body, 0)


def _gather(table: jax.Array, flat_idx: jax.Array) -> jax.Array:
    num_rows, dim = table.shape
    n = flat_idx.shape[0]
    n_pad = ((n + IDX_BLOCK - 1) // IDX_BLOCK) * IDX_BLOCK
    num_blocks = n_pad // IDX_BLOCK
    idx3d = jnp.pad(flat_idx, (0, n_pad - n)).reshape(num_blocks, 1, IDX_BLOCK)
    table3 = table.reshape(num_rows, 1, dim)

    itemsize = table.dtype.itemsize
    cost = pl.CostEstimate(
        flops=0,
        transcendentals=0,
        bytes_accessed=num_rows * dim * itemsize + n_pad * dim * itemsize
        + n_pad * 4,
    )

    out = pl.pallas_call(
        _gather_kernel,
        grid=(num_blocks,),
        in_specs=[
            pl.BlockSpec((1, 1, IDX_BLOCK), lambda i: (i, 0, 0),
                         memory_space=pltpu.SMEM),
            pl.BlockSpec((num_rows, 1, dim), lambda i: (0, 0, 0)),
        ],
        out_specs=pl.BlockSpec((IDX_BLOCK, 1, dim), lambda i: (i, 0, 0)),
        out_shape=jax.ShapeDtypeStruct((n_pad, 1, dim), table.dtype),
        compiler_params=pltpu.CompilerParams(
            dimension_semantics=("parallel",),
            vmem_limit_bytes=32 * 1024 * 1024,
        ),
        cost_estimate=cost,
    )(idx3d, table3)

    return out.reshape(n_pad, dim)[:n]


def kernel(table, num_group):
    num_rows, dim = table.shape

    def _local(t, g):
        flat = jnp.clip(g.reshape(-1).astype(jnp.int32), 0, num_rows - 1)
        return _gather(t, flat)

    devs = jax.devices()
    ids = num_group.reshape(-1, 1) if num_group.ndim < 2 else num_group
    if len(devs) >= 2 and ids.shape[0] % 2 == 0:
        mesh = Mesh(np.array(devs[:2]), ("x",))
        out = shard_map(
            _local, mesh=mesh,
            in_specs=(P(), P("x")), out_specs=P("x"), check_rep=False,
        )(table, ids)
    else:
        out = _local(table, ids)
    return out.reshape(num_group.shape + (dim,))


# speedup vs baseline: 5.7705x; 1.0309x over previous
import numpy as np

import jax
import jax.numpy as jnp
from jax import lax
from jax.experimental import pallas as pl
from jax.experimental.pallas import tpu as pltpu
from jax.experimental.shard_map import shard_map
from jax.sharding import Mesh, PartitionSpec as P

IDX_BLOCK = 8192
UNROLL = 1024


def _gather_kernel(idx_ref, table_ref, out_ref):
    def body(c, carry):
        base = c * UNROLL
        rows = [table_ref[idx_ref[0, 0, base + u]] for u in range(UNROLL)]
        for u in range(UNROLL):
            out_ref[base + u] = rows[u]
        return carry

    lax.fori_loop(0, IDX_BLOCK // UNROLL, body, 0)


def _gather(table: jax.Array, flat_idx: jax.Array) -> jax.Array:
    num_rows, dim = table.shape
    n = flat_idx.shape[0]
    n_pad = ((n + IDX_BLOCK - 1) // IDX_BLOCK) * IDX_BLOCK
    num_blocks = n_pad // IDX_BLOCK
    idx3d = jnp.pad(flat_idx, (0, n_pad - n)).reshape(num_blocks, 1, IDX_BLOCK)
    table3 = table.reshape(num_rows, 1, dim)

    itemsize = table.dtype.itemsize
    cost = pl.CostEstimate(
        flops=0,
        transcendentals=0,
        bytes_accessed=num_rows * dim * itemsize + n_pad * dim * itemsize
        + n_pad * 4,
    )

    out = pl.pallas_call(
        _gather_kernel,
        grid=(num_blocks,),
        in_specs=[
            pl.BlockSpec((1, 1, IDX_BLOCK), lambda i: (i, 0, 0),
                         memory_space=pltpu.SMEM),
            pl.BlockSpec((num_rows, 1, dim), lambda i: (0, 0, 0)),
        ],
        out_specs=pl.BlockSpec((IDX_BLOCK, 1, dim), lambda i: (i, 0, 0)),
        out_shape=jax.ShapeDtypeStruct((n_pad, 1, dim), table.dtype),
        compiler_params=pltpu.CompilerParams(
            dimension_semantics=("parallel",),
            vmem_limit_bytes=32 * 1024 * 1024,
        ),
        cost_estimate=cost,
    )(idx3d, table3)

    return out.reshape(n_pad, dim)[:n]


def kernel(table, num_group):
    num_rows, dim = table.shape

    def _local(t, g):
        flat = jnp.clip(g.reshape(-1).astype(jnp.int32), 0, num_rows - 1)
        return _gather(t, flat)

    devs = jax.devices()
    ids = num_group.reshape(-1, 1) if num_group.ndim < 2 else num_group
    if len(devs) >= 2 and ids.shape[0] % 2 == 0:
        mesh = Mesh(np.array(devs[:2]), ("x",))
        out = shard_map(
            _local, mesh=mesh,
            in_specs=(P(), P("x")), out_specs=P("x"), check_rep=False,
        )(table, ids)
    else:
        out = _local(table, ids)
    return out.reshape(num_group.shape + (dim,))
